# baseline (speedup 1.0000x reference)
import functools

import numpy as np
import jax
import jax.numpy as jnp
from jax import lax
from jax.experimental import pallas as pl
from jax.experimental.pallas import tpu as pltpu
from jax.experimental.pallas import tpu_sc as plsc

_B, _F = 16384, 100
_N = _B * _F
_PROB = 0.15
_NC, _NS, _L = 2, 16, 16
_H = _B // 2
_F0 = [0, 8, 16, 24, 32, 40, 48, 56, 64, 72, 80, 88, 96]
_NBLK = len(_F0)
_BW = [8] * 12 + [4]


def _threefry2x32(k1, k2, x0, x1):
    rot = [[13, 15, 26, 6], [17, 29, 16, 24]]
    u = np.uint32
    ks = [u(k1), u(k2), u(u(k1) ^ u(k2) ^ u(0x1BD11BDA))]
    x0 = x0.astype(np.uint32) + ks[0]
    x1 = x1.astype(np.uint32) + ks[1]

    def rnd(x0, x1, r):
        x0 = x0 + x1
        x1 = (x1 << u(r)) | (x1 >> u(32 - r))
        return x0, x1 ^ x0

    for r in rot[0]:
        x0, x1 = rnd(x0, x1, r)
    x0 = x0 + ks[1]; x1 = x1 + ks[2] + u(1)
    for r in rot[1]:
        x0, x1 = rnd(x0, x1, r)
    x0 = x0 + ks[2]; x1 = x1 + ks[0] + u(2)
    for r in rot[0]:
        x0, x1 = rnd(x0, x1, r)
    x0 = x0 + ks[0]; x1 = x1 + ks[1] + u(3)
    for r in rot[1]:
        x0, x1 = rnd(x0, x1, r)
    x0 = x0 + ks[1]; x1 = x1 + ks[2] + u(4)
    for r in rot[0]:
        x0, x1 = rnd(x0, x1, r)
    x0 = x0 + ks[2]; x1 = x1 + ks[0] + u(5)
    return x0, x1


def _uniform01(key, n):
    i = np.arange(n, dtype=np.uint64)
    c1 = (i >> np.uint64(32)).astype(np.uint32)
    c2 = (i & np.uint64(0xFFFFFFFF)).astype(np.uint32)
    b1, b2 = _threefry2x32(key[0], key[1], c1, c2)
    bits = b1 ^ b2
    f = (((bits >> np.uint32(9)) | np.uint32(0x3F800000)).view(np.float32)
         - np.float32(1.0))
    return np.maximum(np.float32(0.0), f)


def _pad128(n):
    return max(128, -(-n // 128) * 128)


@functools.cache
def _swap_tables():
    b1, b2 = _threefry2x32(np.uint32(0), np.uint32(42),
                           np.zeros(2, np.uint32),
                           np.arange(2, dtype=np.uint32))
    k_mask, k_shift = (b1[0], b2[0]), (b1[1], b2[1])
    mask = _uniform01(k_mask, _N) < np.float32(_PROB)
    row_shift = np.floor(
        _uniform01(k_shift, _N) * np.float32(_B)).astype(np.int32)
    shift = row_shift * (mask.astype(np.int32) * _F)
    idx = np.arange(_N, dtype=np.int32) + shift
    idx = np.where(idx >= _N, idx - _N, idx)

    moved = np.nonzero(idx != np.arange(_N, dtype=np.int32))[0]
    dst_row, dst_col = moved // _F, moved % _F
    src_row = idx[moved] // _F

    def unit_wid(blk, half):
        c = 0 if blk < 7 else 1
        s = (blk - 7 * c) * 2 + half
        return c * 16 + s

    loc_src = [[] for _ in range(32)]
    loc_dst = [[] for _ in range(32)]
    snd_src = [[] for _ in range(32)]
    rcv_dst = [[] for _ in range(32)]
    for b in range(_NBLK):
        f0 = _F0[b]
        sel = (dst_col >= f0) & (dst_col < f0 + _BW[b])
        fi = dst_col[sel] - f0
        dr, sr = dst_row[sel], src_row[sel]
        hd, hs = dr // _H, sr // _H
        psrc = fi * _H + (sr % _H)
        pdst = fi * _H + (dr % _H)
        same = hd == hs
        for half in (0, 1):
            w = unit_wid(b, half)
            m = same & (hd == half)
            loc_src[w] = psrc[m]
            loc_dst[w] = pdst[m]
            ms = (~same) & (hs == half)
            snd_src[w] = psrc[ms]
            rcv_dst[unit_wid(b, 1 - half)] = pdst[ms]

    locpad = _pad128(max(len(a) for a in loc_src))
    sndpad = _pad128(max(max(len(a) for a in snd_src),
                         max(len(a) for a in rcv_dst)))
    loc = np.zeros((32, locpad), np.int64)
    xch = np.zeros((32, sndpad), np.int64)
    meta = np.zeros((32, 128), np.int32)
    for w in range(32):
        ls = np.asarray(loc_src[w], np.int64)
        ld = np.asarray(loc_dst[w], np.int64)
        ss = np.asarray(snd_src[w], np.int64)
        rd = np.asarray(rcv_dst[w], np.int64)
        n = len(ls)
        loc[w, :n] = ((ls >> 13) << 26) | ((ls & (_H - 1)) << 13) | \
            (ld & (_H - 1))
        xch[w, :len(ss)] |= ss
        xch[w, :len(rd)] |= rd << 16
        meta[w, 0:16] = n
        meta[w, 16:32] = len(snd_src[w])
        meta[w, 32:48] = len(rcv_dst[w])
    t = {
        "loc": loc.astype(np.uint32).view(np.int32).reshape(-1),
        "xchg": xch.astype(np.uint32).view(np.int32).reshape(-1),
        "meta": meta.reshape(-1),
        "locpad": locpad, "sndpad": sndpad,
    }
    return t


@functools.cache
def _build_sc_call(locpad, sndpad):
    mesh = plsc.VectorSubcoreMesh(core_axis_name="c", subcore_axis_name="s")

    @functools.partial(
        pl.kernel,
        out_type=jax.ShapeDtypeStruct((104, _B), jnp.float32),
        mesh=mesh,
        compiler_params=pltpu.CompilerParams(
            needs_layout_passes=False, use_tc_tiling_on_sc=True),
        scratch_types=[
            pltpu.VMEM((8, _H), jnp.float32),
            pltpu.VMEM((locpad,), jnp.int32),
            pltpu.VMEM((sndpad,), jnp.int32),
            pltpu.VMEM((locpad,), jnp.float32),
            pltpu.VMEM((sndpad,), jnp.float32),
            pltpu.VMEM((128,), jnp.int32),
            pltpu.VMEM_SHARED((16, sndpad), jnp.float32),
            pltpu.SemaphoreType.DMA,
        ],
    )
    def sc_kernel(xt_hbm, xtail_hbm, loc_hbm, xchg_hbm, meta_hbm, outt_hbm,
                  slab_v, idxa_v, idxb_v, vala_v, valb_v, meta_v, xchg_sh,
                  sem_a):
        c = lax.axis_index("c")
        s = lax.axis_index("s")
        wid = c * 16 + s
        active = s < 14 - 2 * c
        blk = c * 7 + lax.div(s, 2)
        r0 = pl.multiple_of(lax.rem(s, 2) * _H, 128)
        lanes = lax.iota(jnp.int32, 16)

        def unpack_gather16(shift, idx_v, val_v, i, _):
            p = lax.shift_right_logical(idx_v[pl.ds(i * 16, 16)], shift)
            fi = lax.bitwise_and(lax.shift_right_logical(p, 13), 7)
            ri = lax.bitwise_and(p, _H - 1)
            val_v[pl.ds(i * 16, 16)] = plsc.load_gather(slab_v, [fi, ri])
            return _

        def loc_scatter16(nvec, i, _):
            p = idxa_v[pl.ds(i * 16, 16)]
            fi = lax.shift_right_logical(p, 26)
            ri = lax.bitwise_and(p, _H - 1)
            m = (lanes + i * 16) < nvec
            plsc.store_scatter(slab_v, [fi, ri], vala_v[pl.ds(i * 16, 16)],
                               mask=m)
            return _

        def rcv_scatter16(nvec, i, _):
            q = lax.shift_right_logical(idxb_v[pl.ds(i * 16, 16)], 16)
            fi = lax.shift_right_logical(q, 13)
            ri = lax.bitwise_and(q, _H - 1)
            m = (lanes + i * 16) < nvec
            plsc.store_scatter(slab_v, [fi, ri], valb_v[pl.ds(i * 16, 16)],
                               mask=m)
            return _

        @pl.when(active & (blk < 12))
        def _():
            f0 = pl.multiple_of(blk * 8, 8)
            pltpu.make_async_copy(xt_hbm.at[pl.ds(f0, 8), pl.ds(r0, _H)],
                                  slab_v, sem_a).start()

        @pl.when(active & (blk == 12))
        def _():
            pltpu.make_async_copy(xtail_hbm.at[:, pl.ds(r0, _H)],
                                  slab_v.at[pl.ds(0, 4), :], sem_a).start()

        @pl.when(active)
        def _():
            pltpu.sync_copy(meta_hbm.at[pl.ds(wid * 128, 128)], meta_v)
            pltpu.sync_copy(loc_hbm.at[pl.ds(wid * locpad, locpad)], idxa_v)
            pltpu.sync_copy(xchg_hbm.at[pl.ds(wid * sndpad, sndpad)], idxb_v)

        @pl.when(active & (blk < 12))
        def _():
            f0 = pl.multiple_of(blk * 8, 8)
            pltpu.make_async_copy(xt_hbm.at[pl.ds(f0, 8), pl.ds(r0, _H)],
                                  slab_v, sem_a).wait()

        @pl.when(active & (blk == 12))
        def _():
            pltpu.make_async_copy(xtail_hbm.at[:, pl.ds(r0, _H)],
                                  slab_v.at[pl.ds(0, 4), :], sem_a).wait()

        @pl.when(active)
        def _():
            n_loc = meta_v[pl.ds(0, 16)]

            @plsc.parallel_loop(0, locpad // 16, unroll=8)
            def _(i):
                unpack_gather16(13, idxa_v, vala_v, i, 0)

            @plsc.parallel_loop(0, sndpad // 16, unroll=8)
            def _(i):
                unpack_gather16(0, idxb_v, valb_v, i, 0)

            pltpu.sync_copy(valb_v, xchg_sh.at[s])

            @plsc.parallel_loop(0, locpad // 16, unroll=8)
            def _(i):
                loc_scatter16(n_loc, i, 0)

        plsc.subcore_barrier()

        @pl.when(active)
        def _():
            n_rcv = meta_v[pl.ds(32, 16)]
            peer = lax.bitwise_xor(s, 1)
            pltpu.sync_copy(xchg_sh.at[peer], valb_v)

            @plsc.parallel_loop(0, sndpad // 16, unroll=8)
            def _(i):
                rcv_scatter16(n_rcv, i, 0)

            f0 = pl.multiple_of(jnp.where(blk == 12, 96, blk * 8), 8)
            pltpu.sync_copy(slab_v, outt_hbm.at[pl.ds(f0, 8), pl.ds(r0, _H)])

    return sc_kernel


_T = _swap_tables()


def kernel(x):
    xt = jnp.swapaxes(x, 0, 1)
    xtail = jnp.swapaxes(lax.slice(x, (0, 96), (_B, _F)), 0, 1)
    outt = _build_sc_call(_T["locpad"], _T["sndpad"])(
        xt, xtail, _T["loc"], _T["xchg"], _T["meta"])
    return jnp.swapaxes(outt, 0, 1)[:, :_F]


# speedup vs baseline: 1.0976x; 1.0976x over previous
import functools

import numpy as np
import jax
import jax.numpy as jnp
from jax import lax
from jax.experimental import pallas as pl
from jax.experimental.pallas import tpu as pltpu
from jax.experimental.pallas import tpu_sc as plsc

_B, _F = 16384, 100
_N = _B * _F
_PROB = 0.15
_NC, _NS, _L = 2, 16, 16
_H = _B // 2
_F0 = [0, 8, 16, 24, 32, 40, 48, 56, 64, 72, 80, 88, 96]
_NBLK = len(_F0)
_BW = [8] * 12 + [4]


def _threefry2x32(k1, k2, x0, x1):
    rot = [[13, 15, 26, 6], [17, 29, 16, 24]]
    u = np.uint32
    ks = [u(k1), u(k2), u(u(k1) ^ u(k2) ^ u(0x1BD11BDA))]
    x0 = x0.astype(np.uint32) + ks[0]
    x1 = x1.astype(np.uint32) + ks[1]

    def rnd(x0, x1, r):
        x0 = x0 + x1
        x1 = (x1 << u(r)) | (x1 >> u(32 - r))
        return x0, x1 ^ x0

    for r in rot[0]:
        x0, x1 = rnd(x0, x1, r)
    x0 = x0 + ks[1]; x1 = x1 + ks[2] + u(1)
    for r in rot[1]:
        x0, x1 = rnd(x0, x1, r)
    x0 = x0 + ks[2]; x1 = x1 + ks[0] + u(2)
    for r in rot[0]:
        x0, x1 = rnd(x0, x1, r)
    x0 = x0 + ks[0]; x1 = x1 + ks[1] + u(3)
    for r in rot[1]:
        x0, x1 = rnd(x0, x1, r)
    x0 = x0 + ks[1]; x1 = x1 + ks[2] + u(4)
    for r in rot[0]:
        x0, x1 = rnd(x0, x1, r)
    x0 = x0 + ks[2]; x1 = x1 + ks[0] + u(5)
    return x0, x1


def _uniform01(key, n):
    i = np.arange(n, dtype=np.uint64)
    c1 = (i >> np.uint64(32)).astype(np.uint32)
    c2 = (i & np.uint64(0xFFFFFFFF)).astype(np.uint32)
    b1, b2 = _threefry2x32(key[0], key[1], c1, c2)
    bits = b1 ^ b2
    f = (((bits >> np.uint32(9)) | np.uint32(0x3F800000)).view(np.float32)
         - np.float32(1.0))
    return np.maximum(np.float32(0.0), f)


def _pad128(n):
    return max(128, -(-n // 128) * 128)


@functools.cache
def _swap_tables():
    b1, b2 = _threefry2x32(np.uint32(0), np.uint32(42),
                           np.zeros(2, np.uint32),
                           np.arange(2, dtype=np.uint32))
    k_mask, k_shift = (b1[0], b2[0]), (b1[1], b2[1])
    mask = _uniform01(k_mask, _N) < np.float32(_PROB)
    row_shift = np.floor(
        _uniform01(k_shift, _N) * np.float32(_B)).astype(np.int32)
    shift = row_shift * (mask.astype(np.int32) * _F)
    idx = np.arange(_N, dtype=np.int32) + shift
    idx = np.where(idx >= _N, idx - _N, idx)

    moved = np.nonzero(idx != np.arange(_N, dtype=np.int32))[0]
    dst_row, dst_col = moved // _F, moved % _F
    src_row = idx[moved] // _F

    def unit_wid(blk, half):
        c = 0 if blk < 7 else 1
        s = (blk - 7 * c) * 2 + half
        return c * 16 + s

    loc_src = [[] for _ in range(32)]
    loc_dst = [[] for _ in range(32)]
    snd_src = [[] for _ in range(32)]
    rcv_dst = [[] for _ in range(32)]
    for b in range(_NBLK):
        f0 = _F0[b]
        sel = (dst_col >= f0) & (dst_col < f0 + _BW[b])
        fi = dst_col[sel] - f0
        dr, sr = dst_row[sel], src_row[sel]
        hd, hs = dr // _H, sr // _H
        psrc = fi * _H + (sr % _H)
        pdst = fi * _H + (dr % _H)
        same = hd == hs
        for half in (0, 1):
            w = unit_wid(b, half)
            m = same & (hd == half)
            loc_src[w] = psrc[m]
            loc_dst[w] = pdst[m]
            ms = (~same) & (hs == half)
            snd_src[w] = psrc[ms]
            rcv_dst[unit_wid(b, 1 - half)] = pdst[ms]

    locpad = _pad128(max(len(a) for a in loc_src))
    sndpad = _pad128(max(max(len(a) for a in snd_src),
                         max(len(a) for a in rcv_dst)))
    loc = np.zeros((32, locpad), np.int64)
    xch = np.zeros((32, sndpad), np.int64)
    meta = np.zeros((32, 128), np.int32)
    for w in range(32):
        ls = np.asarray(loc_src[w], np.int64)
        ld = np.asarray(loc_dst[w], np.int64)
        ss = np.asarray(snd_src[w], np.int64)
        rd = np.asarray(rcv_dst[w], np.int64)
        n = len(ls)
        loc[w, :n] = ((ls >> 13) << 26) | ((ls & (_H - 1)) << 13) | \
            (ld & (_H - 1))
        xch[w, :len(ss)] |= ss
        xch[w, :len(rd)] |= rd << 16
        meta[w, 0:16] = n
        meta[w, 16:32] = len(snd_src[w])
        meta[w, 32:48] = len(rcv_dst[w])
    tbl = np.concatenate([
        loc.astype(np.uint32).view(np.int32).reshape(-1),
        xch.astype(np.uint32).view(np.int32).reshape(-1),
        meta.reshape(-1),
    ])
    return {"tbl": tbl, "locpad": locpad, "sndpad": sndpad}


@functools.cache
def _build_sc_call(locpad, sndpad):
    mesh = plsc.VectorSubcoreMesh(core_axis_name="c", subcore_axis_name="s")

    @functools.partial(
        pl.kernel,
        out_type=jax.ShapeDtypeStruct((104, _B), jnp.float32),
        mesh=mesh,
        compiler_params=pltpu.CompilerParams(
            needs_layout_passes=False, use_tc_tiling_on_sc=True),
        scratch_types=[
            pltpu.VMEM((8, _H), jnp.float32),
            pltpu.VMEM((locpad,), jnp.int32),
            pltpu.VMEM((sndpad,), jnp.int32),
            pltpu.VMEM((locpad,), jnp.float32),
            pltpu.VMEM((sndpad,), jnp.float32),
            pltpu.VMEM((128,), jnp.int32),
            pltpu.VMEM_SHARED((16, sndpad), jnp.float32),
            pltpu.SemaphoreType.DMA,
        ],
    )
    def sc_kernel(xt_hbm, xtail_hbm, tbl_hbm, outt_hbm,
                  slab_v, idxa_v, idxb_v, vala_v, valb_v, meta_v, xchg_sh,
                  sem_a):
        c = lax.axis_index("c")
        s = lax.axis_index("s")
        wid = c * 16 + s
        active = s < 14 - 2 * c
        blk = c * 7 + lax.div(s, 2)
        r0 = pl.multiple_of(lax.rem(s, 2) * _H, 128)
        lanes = lax.iota(jnp.int32, 16)

        def unpack_gather16(shift, idx_v, val_v, i, _):
            p = lax.shift_right_logical(idx_v[pl.ds(i * 16, 16)], shift)
            fi = lax.bitwise_and(lax.shift_right_logical(p, 13), 7)
            ri = lax.bitwise_and(p, _H - 1)
            val_v[pl.ds(i * 16, 16)] = plsc.load_gather(slab_v, [fi, ri])
            return _

        def loc_scatter16(nvec, i, _):
            p = idxa_v[pl.ds(i * 16, 16)]
            fi = lax.shift_right_logical(p, 26)
            ri = lax.bitwise_and(p, _H - 1)
            m = (lanes + i * 16) < nvec
            plsc.store_scatter(slab_v, [fi, ri], vala_v[pl.ds(i * 16, 16)],
                               mask=m)
            return _

        def rcv_scatter16(nvec, i, _):
            q = lax.shift_right_logical(idxb_v[pl.ds(i * 16, 16)], 16)
            fi = lax.shift_right_logical(q, 13)
            ri = lax.bitwise_and(q, _H - 1)
            m = (lanes + i * 16) < nvec
            plsc.store_scatter(slab_v, [fi, ri], valb_v[pl.ds(i * 16, 16)],
                               mask=m)
            return _

        @pl.when(active & (blk < 12))
        def _():
            f0 = pl.multiple_of(blk * 8, 8)
            pltpu.make_async_copy(xt_hbm.at[pl.ds(f0, 8), pl.ds(r0, _H)],
                                  slab_v, sem_a).start()

        @pl.when(active & (blk == 12))
        def _():
            pltpu.make_async_copy(xtail_hbm.at[:, pl.ds(r0, _H)],
                                  slab_v.at[pl.ds(0, 4), :], sem_a).start()

        @pl.when(active)
        def _():
            xchg_base = 32 * locpad
            meta_base = 32 * (locpad + sndpad)
            pltpu.sync_copy(tbl_hbm.at[pl.ds(meta_base + wid * 128, 128)],
                            meta_v)
            pltpu.sync_copy(tbl_hbm.at[pl.ds(wid * locpad, locpad)], idxa_v)
            pltpu.sync_copy(tbl_hbm.at[pl.ds(xchg_base + wid * sndpad, sndpad)],
                            idxb_v)

        @pl.when(active & (blk < 12))
        def _():
            f0 = pl.multiple_of(blk * 8, 8)
            pltpu.make_async_copy(xt_hbm.at[pl.ds(f0, 8), pl.ds(r0, _H)],
                                  slab_v, sem_a).wait()

        @pl.when(active & (blk == 12))
        def _():
            pltpu.make_async_copy(xtail_hbm.at[:, pl.ds(r0, _H)],
                                  slab_v.at[pl.ds(0, 4), :], sem_a).wait()

        @pl.when(active)
        def _():
            n_loc = meta_v[pl.ds(0, 16)]

            @plsc.parallel_loop(0, locpad // 16, unroll=8)
            def _(i):
                unpack_gather16(13, idxa_v, vala_v, i, 0)

            @plsc.parallel_loop(0, sndpad // 16, unroll=8)
            def _(i):
                unpack_gather16(0, idxb_v, valb_v, i, 0)

            pltpu.sync_copy(valb_v, xchg_sh.at[s])

            @plsc.parallel_loop(0, locpad // 16, unroll=8)
            def _(i):
                loc_scatter16(n_loc, i, 0)

        plsc.subcore_barrier()

        @pl.when(active)
        def _():
            n_rcv = meta_v[pl.ds(32, 16)]
            peer = lax.bitwise_xor(s, 1)
            pltpu.sync_copy(xchg_sh.at[peer], valb_v)

            @plsc.parallel_loop(0, sndpad // 16, unroll=8)
            def _(i):
                rcv_scatter16(n_rcv, i, 0)

            f0 = pl.multiple_of(jnp.where(blk == 12, 96, blk * 8), 8)
            pltpu.sync_copy(slab_v, outt_hbm.at[pl.ds(f0, 8), pl.ds(r0, _H)])

    return sc_kernel


_T = _swap_tables()


def kernel(x):
    xt = jnp.swapaxes(x, 0, 1)
    xtail = jnp.swapaxes(lax.slice(x, (0, 96), (_B, _F)), 0, 1)
    outt = _build_sc_call(_T["locpad"], _T["sndpad"])(xt, xtail, _T["tbl"])
    return jnp.swapaxes(outt, 0, 1)[:, :_F]


# speedup vs baseline: 1.1295x; 1.0291x over previous
import functools

import numpy as np
import jax
import jax.numpy as jnp
from jax import lax
from jax.experimental import pallas as pl
from jax.experimental.pallas import tpu as pltpu
from jax.experimental.pallas import tpu_sc as plsc

_B, _F = 16384, 100
_N = _B * _F
_PROB = 0.15
_NC, _NS, _L = 2, 16, 16
_H = _B // 2
_F0 = [0, 8, 16, 24, 32, 40, 48, 56, 64, 72, 80, 88, 96]
_NBLK = len(_F0)
_BW = [8] * 12 + [4]


def _threefry2x32(k1, k2, x0, x1):
    rot = [[13, 15, 26, 6], [17, 29, 16, 24]]
    u = np.uint32
    ks = [u(k1), u(k2), u(u(k1) ^ u(k2) ^ u(0x1BD11BDA))]
    x0 = x0.astype(np.uint32) + ks[0]
    x1 = x1.astype(np.uint32) + ks[1]

    def rnd(x0, x1, r):
        x0 = x0 + x1
        x1 = (x1 << u(r)) | (x1 >> u(32 - r))
        return x0, x1 ^ x0

    for r in rot[0]:
        x0, x1 = rnd(x0, x1, r)
    x0 = x0 + ks[1]; x1 = x1 + ks[2] + u(1)
    for r in rot[1]:
        x0, x1 = rnd(x0, x1, r)
    x0 = x0 + ks[2]; x1 = x1 + ks[0] + u(2)
    for r in rot[0]:
        x0, x1 = rnd(x0, x1, r)
    x0 = x0 + ks[0]; x1 = x1 + ks[1] + u(3)
    for r in rot[1]:
        x0, x1 = rnd(x0, x1, r)
    x0 = x0 + ks[1]; x1 = x1 + ks[2] + u(4)
    for r in rot[0]:
        x0, x1 = rnd(x0, x1, r)
    x0 = x0 + ks[2]; x1 = x1 + ks[0] + u(5)
    return x0, x1


def _uniform01(key, n):
    i = np.arange(n, dtype=np.uint64)
    c1 = (i >> np.uint64(32)).astype(np.uint32)
    c2 = (i & np.uint64(0xFFFFFFFF)).astype(np.uint32)
    b1, b2 = _threefry2x32(key[0], key[1], c1, c2)
    bits = b1 ^ b2
    f = (((bits >> np.uint32(9)) | np.uint32(0x3F800000)).view(np.float32)
         - np.float32(1.0))
    return np.maximum(np.float32(0.0), f)


def _pad128(n):
    return max(128, -(-n // 128) * 128)


@functools.cache
def _swap_tables():
    b1, b2 = _threefry2x32(np.uint32(0), np.uint32(42),
                           np.zeros(2, np.uint32),
                           np.arange(2, dtype=np.uint32))
    k_mask, k_shift = (b1[0], b2[0]), (b1[1], b2[1])
    mask = _uniform01(k_mask, _N) < np.float32(_PROB)
    row_shift = np.floor(
        _uniform01(k_shift, _N) * np.float32(_B)).astype(np.int32)
    shift = row_shift * (mask.astype(np.int32) * _F)
    idx = np.arange(_N, dtype=np.int32) + shift
    idx = np.where(idx >= _N, idx - _N, idx)

    moved = np.nonzero(idx != np.arange(_N, dtype=np.int32))[0]
    dst_row, dst_col = moved // _F, moved % _F
    src_row = idx[moved] // _F

    def unit_wid(blk, half):
        c = 0 if blk < 7 else 1
        s = (blk - 7 * c) * 2 + half
        return c * 16 + s

    loc_src = [[] for _ in range(32)]
    loc_dst = [[] for _ in range(32)]
    snd_src = [[] for _ in range(32)]
    rcv_dst = [[] for _ in range(32)]
    for b in range(_NBLK):
        f0 = _F0[b]
        sel = (dst_col >= f0) & (dst_col < f0 + _BW[b])
        fi = dst_col[sel] - f0
        dr, sr = dst_row[sel], src_row[sel]
        hd, hs = dr // _H, sr // _H
        psrc = fi * _H + (sr % _H)
        pdst = fi * _H + (dr % _H)
        same = hd == hs
        for half in (0, 1):
            w = unit_wid(b, half)
            m = same & (hd == half)
            loc_src[w] = psrc[m]
            loc_dst[w] = pdst[m]
            ms = (~same) & (hs == half)
            snd_src[w] = psrc[ms]
            rcv_dst[unit_wid(b, 1 - half)] = pdst[ms]

    locpad = _pad128(max(len(a) for a in loc_src))
    sndpad = _pad128(max(max(len(a) for a in snd_src),
                         max(len(a) for a in rcv_dst)))
    loc = np.zeros((32, locpad), np.int64)
    xch = np.zeros((32, sndpad), np.int64)
    meta = np.zeros((32, 128), np.int32)
    for w in range(32):
        ls = np.asarray(loc_src[w], np.int64)
        ld = np.asarray(loc_dst[w], np.int64)
        ss = np.asarray(snd_src[w], np.int64)
        rd = np.asarray(rcv_dst[w], np.int64)
        n = len(ls)
        loc[w, :n] = ((ls >> 13) << 26) | ((ls & (_H - 1)) << 13) | \
            (ld & (_H - 1))
        xch[w, :len(ss)] |= ss
        xch[w, :len(rd)] |= rd << 16
        meta[w, 0:16] = n
        meta[w, 16:32] = len(snd_src[w])
        meta[w, 32:48] = len(rcv_dst[w])
    tbl = np.concatenate([
        loc.astype(np.uint32).view(np.int32).reshape(-1),
        xch.astype(np.uint32).view(np.int32).reshape(-1),
        meta.reshape(-1),
    ])
    return {"tbl": tbl, "locpad": locpad, "sndpad": sndpad}


@functools.cache
def _build_sc_call(locpad, sndpad):
    mesh = plsc.VectorSubcoreMesh(core_axis_name="c", subcore_axis_name="s")

    @functools.partial(
        pl.kernel,
        out_type=jax.ShapeDtypeStruct((104, _B), jnp.float32),
        mesh=mesh,
        compiler_params=pltpu.CompilerParams(
            needs_layout_passes=False, use_tc_tiling_on_sc=True),
        scratch_types=[
            pltpu.VMEM((8, _H), jnp.float32),
            pltpu.VMEM((locpad,), jnp.int32),
            pltpu.VMEM((sndpad,), jnp.int32),
            pltpu.VMEM((locpad,), jnp.float32),
            pltpu.VMEM((sndpad,), jnp.float32),
            pltpu.VMEM((128,), jnp.int32),
            pltpu.VMEM_SHARED((16, sndpad), jnp.float32),
            pltpu.SemaphoreType.DMA,
        ],
    )
    def sc_kernel(xt_hbm, tbl_hbm, outt_hbm,
                  slab_v, idxa_v, idxb_v, vala_v, valb_v, meta_v, xchg_sh,
                  sem_a):
        c = lax.axis_index("c")
        s = lax.axis_index("s")
        wid = c * 16 + s
        active = s < 14 - 2 * c
        blk = c * 7 + lax.div(s, 2)
        r0 = pl.multiple_of(lax.rem(s, 2) * _H, 128)
        lanes = lax.iota(jnp.int32, 16)

        def unpack_gather16(shift, idx_v, val_v, i, _):
            p = lax.shift_right_logical(idx_v[pl.ds(i * 16, 16)], shift)
            fi = lax.bitwise_and(lax.shift_right_logical(p, 13), 7)
            ri = lax.bitwise_and(p, _H - 1)
            val_v[pl.ds(i * 16, 16)] = plsc.load_gather(slab_v, [fi, ri])
            return _

        def loc_scatter16(nvec, i, _):
            p = idxa_v[pl.ds(i * 16, 16)]
            fi = lax.shift_right_logical(p, 26)
            ri = lax.bitwise_and(p, _H - 1)
            m = (lanes + i * 16) < nvec
            plsc.store_scatter(slab_v, [fi, ri], vala_v[pl.ds(i * 16, 16)],
                               mask=m)
            return _

        def rcv_scatter16(nvec, i, _):
            q = lax.shift_right_logical(idxb_v[pl.ds(i * 16, 16)], 16)
            fi = lax.shift_right_logical(q, 13)
            ri = lax.bitwise_and(q, _H - 1)
            m = (lanes + i * 16) < nvec
            plsc.store_scatter(slab_v, [fi, ri], valb_v[pl.ds(i * 16, 16)],
                               mask=m)
            return _

        @pl.when(active & (blk < 12))
        def _():
            f0 = pl.multiple_of(blk * 8, 8)
            pltpu.make_async_copy(xt_hbm.at[pl.ds(f0, 8), pl.ds(r0, _H)],
                                  slab_v, sem_a).start()

        @pl.when(active & (blk == 12))
        def _():
            pltpu.make_async_copy(xt_hbm.at[pl.ds(96, 4), pl.ds(r0, _H)],
                                  slab_v.at[pl.ds(0, 4), :], sem_a).start()

        @pl.when(active)
        def _():
            xchg_base = 32 * locpad
            meta_base = 32 * (locpad + sndpad)
            pltpu.sync_copy(tbl_hbm.at[pl.ds(meta_base + wid * 128, 128)],
                            meta_v)
            pltpu.sync_copy(tbl_hbm.at[pl.ds(wid * locpad, locpad)], idxa_v)
            pltpu.sync_copy(tbl_hbm.at[pl.ds(xchg_base + wid * sndpad, sndpad)],
                            idxb_v)

        @pl.when(active & (blk < 12))
        def _():
            f0 = pl.multiple_of(blk * 8, 8)
            pltpu.make_async_copy(xt_hbm.at[pl.ds(f0, 8), pl.ds(r0, _H)],
                                  slab_v, sem_a).wait()

        @pl.when(active & (blk == 12))
        def _():
            pltpu.make_async_copy(xt_hbm.at[pl.ds(96, 4), pl.ds(r0, _H)],
                                  slab_v.at[pl.ds(0, 4), :], sem_a).wait()

        @pl.when(active)
        def _():
            n_loc = meta_v[pl.ds(0, 16)]

            @plsc.parallel_loop(0, locpad // 16, unroll=8)
            def _(i):
                unpack_gather16(13, idxa_v, vala_v, i, 0)

            @plsc.parallel_loop(0, sndpad // 16, unroll=8)
            def _(i):
                unpack_gather16(0, idxb_v, valb_v, i, 0)

            pltpu.sync_copy(valb_v, xchg_sh.at[s])

            @plsc.parallel_loop(0, locpad // 16, unroll=8)
            def _(i):
                loc_scatter16(n_loc, i, 0)

        plsc.subcore_barrier()

        @pl.when(active)
        def _():
            n_rcv = meta_v[pl.ds(32, 16)]
            peer = lax.bitwise_xor(s, 1)
            pltpu.sync_copy(xchg_sh.at[peer], valb_v)

            @plsc.parallel_loop(0, sndpad // 16, unroll=8)
            def _(i):
                rcv_scatter16(n_rcv, i, 0)

            f0 = pl.multiple_of(jnp.where(blk == 12, 96, blk * 8), 8)
            pltpu.sync_copy(slab_v, outt_hbm.at[pl.ds(f0, 8), pl.ds(r0, _H)])

    return sc_kernel


_T = _swap_tables()


def kernel(x):
    xt = jnp.swapaxes(x, 0, 1)
    outt = _build_sc_call(_T["locpad"], _T["sndpad"])(xt, _T["tbl"])
    return jnp.swapaxes(outt, 0, 1)[:, :_F]
